# baseline (speedup 1.0000x reference)
import jax
import jax.numpy as jnp
from jax.experimental import pallas as pl
from jax.experimental.pallas import tpu as pltpu

N = 10000
D = 128
BM = 400
NI = N // BM


def _deg_h_kernel(a_ref, f_ref, w_ref, deg_ref, h_ref):
    deg = jnp.sum(a_ref[...], axis=1, keepdims=True)
    deg_ref[...] = deg
    norm = jnp.where(deg > 0.0, jax.lax.rsqrt(deg), 0.0)
    fw = jax.lax.dot_general(
        f_ref[...], w_ref[...],
        dimension_numbers=(((1,), (1,)), ((), ())),
        preferred_element_type=jnp.float32,
    )
    h_ref[...] = fw * norm


def _spmm_kernel(a_ref, h_ref, deg_ref, b_ref, out_ref):
    deg = deg_ref[...]
    norm = jnp.where(deg > 0.0, jax.lax.rsqrt(deg), 0.0)
    acc = jnp.dot(a_ref[...], h_ref[...], preferred_element_type=jnp.float32)
    out_ref[...] = acc * norm + b_ref[...]


def kernel(Adjacency, Features, W, b):
    assert Adjacency.shape == (N, N)
    assert Features.shape == (N, D)

    deg, h = pl.pallas_call(
        _deg_h_kernel,
        grid=(NI,),
        in_specs=[
            pl.BlockSpec((BM, N), lambda i: (i, 0)),
            pl.BlockSpec((BM, D), lambda i: (i, 0)),
            pl.BlockSpec((D, D), lambda i: (0, 0)),
        ],
        out_specs=[
            pl.BlockSpec((BM, 1), lambda i: (i, 0)),
            pl.BlockSpec((BM, D), lambda i: (i, 0)),
        ],
        out_shape=[
            jax.ShapeDtypeStruct((N, 1), jnp.float32),
            jax.ShapeDtypeStruct((N, D), jnp.float32),
        ],
        compiler_params=pltpu.CompilerParams(
            dimension_semantics=("parallel",)),
    )(Adjacency, Features, W)

    out = pl.pallas_call(
        _spmm_kernel,
        grid=(NI,),
        in_specs=[
            pl.BlockSpec((BM, N), lambda i: (i, 0)),
            pl.BlockSpec((N, D), lambda i: (0, 0)),
            pl.BlockSpec((BM, 1), lambda i: (i, 0)),
            pl.BlockSpec((1, D), lambda i: (0, 0)),
        ],
        out_specs=pl.BlockSpec((BM, D), lambda i: (i, 0)),
        out_shape=jax.ShapeDtypeStruct((N, D), jnp.float32),
        compiler_params=pltpu.CompilerParams(
            dimension_semantics=("parallel",)),
    )(Adjacency, h, deg, b.reshape(1, D))
    return out


# speedup vs baseline: 1.0440x; 1.0440x over previous
import jax
import jax.numpy as jnp
from jax.experimental import pallas as pl
from jax.experimental.pallas import tpu as pltpu

N = 10000
D = 128
BM = 400
NI = N // BM


def _gcn_kernel(a_ref, f_ref, w_ref, b_ref, out_ref, h_scr, norm_scr):
    phase = pl.program_id(0)
    i = pl.program_id(1)

    @pl.when(phase == 0)
    def _deg_h():
        deg = jnp.sum(a_ref[...], axis=1, keepdims=True)
        norm = jnp.where(deg > 0.0, jax.lax.rsqrt(deg), 0.0)
        norm_scr[pl.ds(i * BM, BM), :] = norm
        fw = jax.lax.dot_general(
            f_ref[...], w_ref[...],
            dimension_numbers=(((1,), (1,)), ((), ())),
            preferred_element_type=jnp.float32,
        )
        h_scr[pl.ds(i * BM, BM), :] = fw * norm

    @pl.when(phase == 1)
    def _spmm():
        acc = jnp.dot(a_ref[...], h_scr[...],
                      preferred_element_type=jnp.float32)
        out_ref[...] = acc * norm_scr[pl.ds(i * BM, BM), :] + b_ref[...]


def kernel(Adjacency, Features, W, b):
    assert Adjacency.shape == (N, N)
    assert Features.shape == (N, D)

    out = pl.pallas_call(
        _gcn_kernel,
        grid=(2, NI),
        in_specs=[
            pl.BlockSpec((BM, N), lambda p, i: (i, 0)),
            pl.BlockSpec((BM, D), lambda p, i: (i, 0)),
            pl.BlockSpec((D, D), lambda p, i: (0, 0)),
            pl.BlockSpec((1, D), lambda p, i: (0, 0)),
        ],
        out_specs=pl.BlockSpec((BM, D), lambda p, i: (p * i, 0)),
        out_shape=jax.ShapeDtypeStruct((N, D), jnp.float32),
        scratch_shapes=[
            pltpu.VMEM((N, D), jnp.float32),
            pltpu.VMEM((N, 1), jnp.float32),
        ],
        compiler_params=pltpu.CompilerParams(
            dimension_semantics=("arbitrary", "arbitrary")),
    )(Adjacency, Features, W, b.reshape(1, D))
    return out


# speedup vs baseline: 1.0482x; 1.0041x over previous
import jax
import jax.numpy as jnp
from jax.experimental import pallas as pl
from jax.experimental.pallas import tpu as pltpu

N = 10000
D = 128
BM = 200
NI = N // BM


def _gcn_kernel(a_ref, f_ref, w_ref, b_ref, out_ref, h_scr, norm_scr):
    phase = pl.program_id(0)
    i = pl.program_id(1)

    @pl.when(phase == 0)
    def _deg_h():
        deg = jnp.sum(a_ref[...], axis=1, keepdims=True)
        norm = jnp.where(deg > 0.0, jax.lax.rsqrt(deg), 0.0)
        norm_scr[pl.ds(i * BM, BM), :] = norm
        fw = jax.lax.dot_general(
            f_ref[...], w_ref[...],
            dimension_numbers=(((1,), (1,)), ((), ())),
            preferred_element_type=jnp.float32,
        )
        h_scr[pl.ds(i * BM, BM), :] = fw * norm

    @pl.when(phase == 1)
    def _spmm():
        acc = jnp.dot(a_ref[...], h_scr[...],
                      preferred_element_type=jnp.float32)
        out_ref[...] = acc * norm_scr[pl.ds(i * BM, BM), :] + b_ref[...]


def kernel(Adjacency, Features, W, b):
    assert Adjacency.shape == (N, N)
    assert Features.shape == (N, D)

    out = pl.pallas_call(
        _gcn_kernel,
        grid=(2, NI),
        in_specs=[
            pl.BlockSpec((BM, N), lambda p, i: (i, 0)),
            pl.BlockSpec((BM, D), lambda p, i: (i, 0)),
            pl.BlockSpec((D, D), lambda p, i: (0, 0)),
            pl.BlockSpec((1, D), lambda p, i: (0, 0)),
        ],
        out_specs=pl.BlockSpec((BM, D), lambda p, i: (p * i, 0)),
        out_shape=jax.ShapeDtypeStruct((N, D), jnp.float32),
        scratch_shapes=[
            pltpu.VMEM((N, D), jnp.float32),
            pltpu.VMEM((N, 1), jnp.float32),
        ],
        compiler_params=pltpu.CompilerParams(
            dimension_semantics=("arbitrary", "arbitrary")),
    )(Adjacency, Features, W, b.reshape(1, D))
    return out
